# baseline (speedup 1.0000x reference)
import functools

import jax
import jax.numpy as jnp
from jax import lax
from jax.experimental import pallas as pl
from jax.experimental.pallas import tpu as pltpu
from jax.experimental.pallas import tpu_sc as plsc

NUM_LEVELS = 100000
TPAD = NUM_LEVELS + 16
H = 40000

NC = 2
NS = 16
L = 16
NW = NC * NS
BC = 128


@functools.lru_cache(maxsize=None)
def _build(C: int, R: int):
    assert R % (NW * BC) == 0
    cols_w = R // NW
    n_blk = cols_w // BC

    mesh = plsc.VectorSubcoreMesh(core_axis_name="c", subcore_axis_name="s")

    @functools.partial(
        pl.kernel,
        mesh=mesh,
        compiler_params=pltpu.CompilerParams(needs_layout_passes=False),
        out_type=jax.ShapeDtypeStruct((C, R), jnp.float32),
        scratch_types=[
            pltpu.VMEM_SHARED((NUM_LEVELS,), jnp.float32),
            pltpu.VMEM((TPAD,), jnp.float32),
            pltpu.VMEM((C, BC), jnp.int32),
            pltpu.VMEM((C, BC), jnp.int32),
            pltpu.VMEM((C, BC), jnp.float32),
            pltpu.VMEM((C, BC), jnp.float32),
            pltpu.VMEM((L,), jnp.float32),
            pltpu.SemaphoreType.DMA,
            pltpu.SemaphoreType.DMA,
            pltpu.SemaphoreType.DMA,
            pltpu.SemaphoreType.DMA,
            pltpu.SemaphoreType.DMA,
        ],
    )
    def sc_gather(fv_hbm, table_hbm, int_hbm, out_hbm,
                  table_sh, table_v, ib0, ib1, ob0, ob1, int_v,
                  sem_t, sem_x, sem_i0, sem_i1, sem_o):
        cid = lax.axis_index("c")
        sid = lax.axis_index("s")
        wid = sid * NC + cid
        base = wid * cols_w
        ibufs = (ib0, ib1)
        obufs = (ob0, ob1)
        isems = (sem_i0, sem_i1)

        def idx_dma(b):
            return pltpu.async_copy(
                fv_hbm.at[:, pl.ds(base + b * BC, BC)], ibufs[b % 2],
                isems[b % 2])

        cp_h = pltpu.async_copy(
            table_hbm.at[pl.ds(0, H)], table_v.at[pl.ds(0, H)], sem_t)
        icps = {0: idx_dma(0), 1: idx_dma(1)}
        cp_s = pltpu.async_copy(int_hbm, int_v, sem_t)

        @pl.when(sid == 0)
        def _():
            pltpu.sync_copy(table_hbm, table_sh)

        plsc.subcore_barrier()
        cp_x = pltpu.async_copy(
            table_sh.at[pl.ds(H, NUM_LEVELS - H)],
            table_v.at[pl.ds(H, NUM_LEVELS - H)], sem_x)
        cp_h.wait()
        cp_x.wait()
        table_v[pl.ds(NUM_LEVELS, L)] = jnp.zeros((L,), jnp.float32)
        cp_s.wait()
        inter = int_v[...]

        ocps = {}
        for b in range(n_blk):
            icps[b].wait()
            if b >= 2:
                ocps[b - 2].wait()
            idx_v = ibufs[b % 2]
            out_v = obufs[b % 2]

            @plsc.parallel_loop(0, C * (BC // L), unroll=4)
            def body(i):
                r = i // (BC // L)
                s = pl.ds((i % (BC // L)) * L, L)
                idx = idx_v[r, s]
                valid = (idx >= 0) & (idx < NUM_LEVELS)
                idx2 = jnp.where(valid, idx, NUM_LEVELS)
                out_v[r, s] = plsc.load_gather(table_v, [idx2]) + inter

            ocps[b] = pltpu.async_copy(
                out_v, out_hbm.at[:, pl.ds(base + b * BC, BC)], sem_o)
            if b + 2 < n_blk:
                icps[b + 2] = idx_dma(b + 2)

        ocps[n_blk - 2].wait()
        ocps[n_blk - 1].wait()

    return sc_gather


def kernel(feature_vals, re_loc, intercept):
    R, C = feature_vals.shape
    fvT = feature_vals.astype(jnp.int32).T
    ivec = jnp.full((L,), intercept, jnp.float32)
    outT = _build(C, R)(fvT, re_loc, ivec)
    return outT.T


# speedup vs baseline: 1.0329x; 1.0329x over previous
import functools

import jax
import jax.numpy as jnp
from jax import lax
from jax.experimental import pallas as pl
from jax.experimental.pallas import tpu as pltpu
from jax.experimental.pallas import tpu_sc as plsc

NUM_LEVELS = 100000
TPAD = NUM_LEVELS + 16
H = 40000

NC = 2
NS = 16
L = 16
NW = NC * NS
BC = 128


@functools.lru_cache(maxsize=None)
def _build(C: int, R: int):
    assert R % (NW * BC) == 0
    cols_w = R // NW
    n_blk = cols_w // BC

    mesh = plsc.VectorSubcoreMesh(core_axis_name="c", subcore_axis_name="s")

    @functools.partial(
        pl.kernel,
        mesh=mesh,
        compiler_params=pltpu.CompilerParams(needs_layout_passes=False),
        out_type=jax.ShapeDtypeStruct((C, R), jnp.float32),
        scratch_types=[
            pltpu.VMEM_SHARED((NUM_LEVELS,), jnp.float32),
            pltpu.VMEM((TPAD,), jnp.float32),
            pltpu.VMEM((C, BC), jnp.int32),
            pltpu.VMEM((C, BC), jnp.int32),
            pltpu.VMEM((C, BC), jnp.float32),
            pltpu.VMEM((C, BC), jnp.float32),
            pltpu.VMEM((L,), jnp.float32),
            pltpu.SemaphoreType.DMA,
            pltpu.SemaphoreType.DMA,
            pltpu.SemaphoreType.DMA,
            pltpu.SemaphoreType.DMA,
            pltpu.SemaphoreType.DMA,
        ],
    )
    def sc_gather(fv_hbm, table_hbm, int_hbm, out_hbm,
                  table_sh, table_v, ib0, ib1, ob0, ob1, int_v,
                  sem_t, sem_x, sem_i0, sem_i1, sem_o):
        cid = lax.axis_index("c")
        sid = lax.axis_index("s")
        wid = sid * NC + cid
        base = wid * cols_w
        ibufs = (ib0, ib1)
        obufs = (ob0, ob1)
        isems = (sem_i0, sem_i1)

        def idx_dma(b):
            return pltpu.async_copy(
                fv_hbm.at[:, pl.ds(base + b * BC, BC)], ibufs[b % 2],
                isems[b % 2])

        icps = {0: idx_dma(0), 1: idx_dma(1)}
        cp_s = pltpu.async_copy(int_hbm, int_v, sem_t)

        @pl.when(sid == 0)
        def _():
            pltpu.sync_copy(table_hbm, table_sh)

        plsc.subcore_barrier()
        cp_x = pltpu.async_copy(
            table_sh, table_v.at[pl.ds(0, NUM_LEVELS)], sem_x)
        cp_x.wait()
        table_v[pl.ds(NUM_LEVELS, L)] = jnp.zeros((L,), jnp.float32)
        cp_s.wait()
        inter = int_v[...]

        ocps = {}
        for b in range(n_blk):
            icps[b].wait()
            if b >= 2:
                ocps[b - 2].wait()
            idx_v = ibufs[b % 2]
            out_v = obufs[b % 2]

            @plsc.parallel_loop(0, C * (BC // L), unroll=4)
            def body(i):
                r = i // (BC // L)
                s = pl.ds((i % (BC // L)) * L, L)
                idx = idx_v[r, s]
                valid = (idx >= 0) & (idx < NUM_LEVELS)
                idx2 = jnp.where(valid, idx, NUM_LEVELS)
                out_v[r, s] = plsc.load_gather(table_v, [idx2]) + inter

            ocps[b] = pltpu.async_copy(
                out_v, out_hbm.at[:, pl.ds(base + b * BC, BC)], sem_o)
            if b + 2 < n_blk:
                icps[b + 2] = idx_dma(b + 2)

        ocps[n_blk - 2].wait()
        ocps[n_blk - 1].wait()

    return sc_gather


def kernel(feature_vals, re_loc, intercept):
    R, C = feature_vals.shape
    fvT = feature_vals.astype(jnp.int32).T
    ivec = jnp.full((L,), intercept, jnp.float32)
    outT = _build(C, R)(fvT, re_loc, ivec)
    return outT.T


# speedup vs baseline: 1.0520x; 1.0185x over previous
import functools

import jax
import jax.numpy as jnp
from jax import lax
from jax.experimental import pallas as pl
from jax.experimental.pallas import tpu as pltpu
from jax.experimental.pallas import tpu_sc as plsc

NUM_LEVELS = 100000
TPAD = NUM_LEVELS + 16

NC = 2
NS = 16
L = 16
NW = NC * NS
BC = 256


@functools.lru_cache(maxsize=None)
def _build(C: int, R: int):
    assert R % (NW * BC) == 0
    cols_w = R // NW
    n_blk = cols_w // BC

    mesh = plsc.VectorSubcoreMesh(core_axis_name="c", subcore_axis_name="s")

    @functools.partial(
        pl.kernel,
        mesh=mesh,
        compiler_params=pltpu.CompilerParams(needs_layout_passes=False),
        out_type=jax.ShapeDtypeStruct((C, R), jnp.float32),
        scratch_types=[
            pltpu.VMEM_SHARED((NUM_LEVELS,), jnp.float32),
            pltpu.VMEM((TPAD,), jnp.float32),
            pltpu.VMEM((C, BC), jnp.int32),
            pltpu.VMEM((C, BC), jnp.int32),
            pltpu.VMEM((C, BC), jnp.float32),
            pltpu.VMEM((L,), jnp.float32),
            pltpu.SemaphoreType.DMA,
            pltpu.SemaphoreType.DMA,
        ],
    )
    def sc_gather(fv_hbm, table_hbm, int_hbm, out_hbm,
                  table_sh, table_v, idx_a, idx_b, out_v, int_v, sem_t, sem_i):
        cid = lax.axis_index("c")
        sid = lax.axis_index("s")
        wid = sid * NC + cid
        base = wid * cols_w
        idx_bufs = (idx_a, idx_b)
        cps = [
            pltpu.async_copy(
                fv_hbm.at[:, pl.ds(base + b * BC, BC)], idx_bufs[b], sem_i)
            for b in range(n_blk)
        ]
        cp_s = pltpu.async_copy(int_hbm, int_v, sem_t)

        @pl.when(sid == 0)
        def _():
            pltpu.sync_copy(table_hbm, table_sh)

        plsc.subcore_barrier()
        pltpu.sync_copy(table_sh, table_v.at[pl.ds(0, NUM_LEVELS)])
        table_v[pl.ds(NUM_LEVELS, L)] = jnp.zeros((L,), jnp.float32)
        cp_s.wait()
        inter = int_v[...]

        for b in range(n_blk):
            cps[b].wait()
            idx_v = idx_bufs[b]

            @plsc.parallel_loop(0, C * (BC // L), unroll=4)
            def body(i):
                r = i // (BC // L)
                s = pl.ds((i % (BC // L)) * L, L)
                idx = idx_v[r, s]
                valid = (idx >= 0) & (idx < NUM_LEVELS)
                idx2 = jnp.where(valid, idx, NUM_LEVELS)
                out_v[r, s] = plsc.load_gather(table_v, [idx2]) + inter

            pltpu.sync_copy(out_v, out_hbm.at[:, pl.ds(base + b * BC, BC)])

    return sc_gather


def kernel(feature_vals, re_loc, intercept):
    R, C = feature_vals.shape
    fvT = feature_vals.astype(jnp.int32).T
    ivec = jnp.full((L,), intercept, jnp.float32)
    outT = _build(C, R)(fvT, re_loc, ivec)
    return outT.T


# speedup vs baseline: 1.0621x; 1.0096x over previous
import functools

import jax
import jax.numpy as jnp
from jax import lax
from jax.experimental import pallas as pl
from jax.experimental.pallas import tpu as pltpu
from jax.experimental.pallas import tpu_sc as plsc

NUM_LEVELS = 100000
TPAD = NUM_LEVELS + 16

NC = 2
NS = 16
L = 16
NW = NC * NS
BC = 256


@functools.lru_cache(maxsize=None)
def _build(C: int, R: int):
    assert R % (NW * BC) == 0
    cols_w = R // NW
    n_blk = cols_w // BC

    mesh = plsc.VectorSubcoreMesh(core_axis_name="c", subcore_axis_name="s")

    @functools.partial(
        pl.kernel,
        mesh=mesh,
        compiler_params=pltpu.CompilerParams(needs_layout_passes=False),
        out_type=jax.ShapeDtypeStruct((C, R), jnp.float32),
        scratch_types=[
            pltpu.VMEM_SHARED((NUM_LEVELS,), jnp.float32),
            pltpu.VMEM((TPAD,), jnp.float32),
            pltpu.VMEM((C, BC), jnp.int32),
            pltpu.VMEM((C, BC), jnp.int32),
            pltpu.VMEM((C, BC), jnp.float32),
            pltpu.VMEM((L,), jnp.float32),
            pltpu.SemaphoreType.DMA,
            pltpu.SemaphoreType.DMA,
        ],
    )
    def sc_gather(fv_hbm, table_hbm, int_hbm, out_hbm,
                  table_sh, table_v, idx_a, idx_b, out_v, int_v, sem_t, sem_i):
        cid = lax.axis_index("c")
        sid = lax.axis_index("s")
        wid = sid * NC + cid
        base = wid * cols_w
        idx_bufs = (idx_a, idx_b)
        cps = [
            pltpu.async_copy(
                fv_hbm.at[:, pl.ds(base + b * BC, BC)], idx_bufs[b], sem_i)
            for b in range(n_blk)
        ]
        cp_s = pltpu.async_copy(int_hbm, int_v.at[pl.ds(0, 1)], sem_t)

        @pl.when(sid == 0)
        def _():
            pltpu.sync_copy(table_hbm, table_sh)

        plsc.subcore_barrier()
        pltpu.sync_copy(table_sh, table_v.at[pl.ds(0, NUM_LEVELS)])
        table_v[pl.ds(NUM_LEVELS, L)] = jnp.zeros((L,), jnp.float32)
        cp_s.wait()
        inter = plsc.load_gather(int_v, [jnp.zeros((L,), jnp.int32)])

        for b in range(n_blk):
            cps[b].wait()
            idx_v = idx_bufs[b]

            @plsc.parallel_loop(0, C * (BC // L), unroll=4)
            def body(i):
                r = i // (BC // L)
                s = pl.ds((i % (BC // L)) * L, L)
                idx = idx_v[r, s]
                valid = (idx >= 0) & (idx < NUM_LEVELS)
                idx2 = jnp.where(valid, idx, NUM_LEVELS)
                out_v[r, s] = plsc.load_gather(table_v, [idx2]) + inter

            pltpu.sync_copy(out_v, out_hbm.at[:, pl.ds(base + b * BC, BC)])

    return sc_gather


def kernel(feature_vals, re_loc, intercept):
    R, C = feature_vals.shape
    fvT = feature_vals.astype(jnp.int32).T
    ivec = jnp.reshape(intercept, (1,)).astype(jnp.float32)
    outT = _build(C, R)(fvT, re_loc, ivec)
    return outT.T
